# Initial kernel scaffold; baseline (speedup 1.0000x reference)
import jax
import jax.numpy as jnp
from jax.experimental import pallas as pl


def kernel(x, edge_index, W1, b1, W2, b2):
    raise NotImplementedError("write your pallas kernel here")


# speedup vs baseline: 6.2598x; 6.2598x over previous
import functools

import jax
import jax.numpy as jnp
from jax import lax
from jax.experimental import pallas as pl
from jax.experimental.pallas import tpu as pltpu
from jax.experimental.pallas import tpu_sc as plsc

N = 10000
E = 320000
D = 128

NC, NS, L = 2, 16, 16
NW = NC * NS
EPW = E // NW
CH = 128
NFULL = EPW // CH
REM = EPW - NFULL * CH
ZR = 80


def _segment_sum_sc(x, src, dst):
    mesh = plsc.VectorSubcoreMesh(
        core_axis_name="c", subcore_axis_name="s", num_cores=NC, num_subcores=NS
    )

    @functools.partial(
        pl.kernel,
        out_type=jax.ShapeDtypeStruct((NC, N, D), jnp.float32),
        mesh=mesh,
        scratch_types=[
            pltpu.VMEM((CH,), jnp.int32),
            pltpu.VMEM((CH,), jnp.int32),
            pltpu.VMEM((CH, D), jnp.float32),
            pltpu.VMEM((REM,), jnp.int32),
            pltpu.VMEM((REM,), jnp.int32),
            pltpu.VMEM((REM, D), jnp.float32),
            pltpu.VMEM((ZR, D), jnp.float32),
            pltpu.VMEM_SHARED((N, D), jnp.float32),
            pltpu.SemaphoreType.DMA,
        ],
    )
    def k(x_hbm, src_hbm, dst_hbm, out_hbm,
          sidx, didx, rows, sidx_r, didx_r, rows_r, zbuf, acc, sem):
        cid = lax.axis_index("c")
        sid = lax.axis_index("s")
        wid = sid * NC + cid
        base = wid * EPW
        nchunks = jnp.where(sid == NS - 1, (N // ZR) - 8 * (NS - 1), 8)

        zv = jnp.zeros((L,), jnp.float32)

        def zero_body(i, _):
            r = i // (D // L)
            col = (i % (D // L)) * L
            zbuf[r, pl.ds(col, L)] = zv
            return 0

        lax.fori_loop(0, ZR * (D // L), zero_body, 0)

        def init_body(j, _):
            r0 = pl.multiple_of((sid * 8 + j) * ZR, 8)
            pltpu.sync_copy(zbuf, acc.at[pl.ds(r0, ZR)])
            return 0

        lax.fori_loop(0, nchunks, init_body, 0)
        plsc.subcore_barrier()

        def chunk(off, si, di, rw):
            pltpu.sync_copy(src_hbm.at[off], si)
            pltpu.sync_copy(dst_hbm.at[off], di)
            pltpu.async_copy(x_hbm.at[si], rw, sem).wait()
            pltpu.sync_copy(rw, acc.at[di], add=True)

        def chunk_body(c, _):
            chunk(pl.ds(base + c * CH, CH), sidx, didx, rows)
            return 0

        lax.fori_loop(0, NFULL, chunk_body, 0)
        chunk(pl.ds(base + NFULL * CH, REM), sidx_r, didx_r, rows_r)
        plsc.subcore_barrier()

        def drain_body(j, _):
            r0 = pl.multiple_of((sid * 8 + j) * ZR, 8)
            pltpu.sync_copy(acc.at[pl.ds(r0, ZR)], zbuf)
            pltpu.sync_copy(zbuf, out_hbm.at[cid, pl.ds(r0, ZR)])
            return 0

        lax.fori_loop(0, nchunks, drain_body, 0)

    return k(x, src, dst)


def _linear_kernel(act):
    def body(x_ref, p0_ref, p1_ref, w_ref, b_ref, o_ref):
        h = x_ref[...] + p0_ref[...] + p1_ref[...]
        z = jnp.dot(h, w_ref[...], preferred_element_type=jnp.float32)
        z = z + b_ref[...]
        if act == "relu":
            o_ref[...] = jnp.maximum(z, 0.0)
        else:
            m = jnp.max(z, axis=1, keepdims=True)
            e = jnp.exp(z - m)
            s = jnp.sum(e, axis=1, keepdims=True)
            o_ref[...] = (z - m) - jnp.log(s)

    return body


def _linear(x, parts, wt, b, act):
    BLK = 1000
    grid = (N // BLK,)
    row_spec = pl.BlockSpec((BLK, D), lambda i: (i, 0))
    full_spec = pl.BlockSpec((D, D), lambda i: (0, 0))
    bias_spec = pl.BlockSpec((1, D), lambda i: (0, 0))
    return pl.pallas_call(
        _linear_kernel(act),
        grid=grid,
        in_specs=[row_spec, row_spec, row_spec, full_spec, bias_spec],
        out_specs=row_spec,
        out_shape=jax.ShapeDtypeStruct((N, D), jnp.float32),
    )(x, parts[0], parts[1], wt, b)


def kernel(x, edge_index, W1, b1, W2, b2):
    src = edge_index[0]
    dst = edge_index[1]
    agg1 = _segment_sum_sc(x, src, dst)
    h1 = _linear(x, agg1, W1.T, b1.reshape(1, D), "relu")
    agg2 = _segment_sum_sc(h1, src, dst)
    return _linear(h1, agg2, W2.T, b2.reshape(1, D), "logsoftmax")


# speedup vs baseline: 12.2715x; 1.9604x over previous
import functools

import jax
import jax.numpy as jnp
from jax import lax
from jax.experimental import pallas as pl
from jax.experimental.pallas import tpu as pltpu
from jax.experimental.pallas import tpu_sc as plsc

N = 10000
E = 320000
D = 128

NC, NS, L = 2, 16, 16
NW = NC * NS
CH = 128
NCHUNKS = E // CH
CPW = NCHUNKS // NW
NLEFT = NCHUNKS - CPW * NW
ZR = 80


def _segment_sum_sc(x, src2, dst2):
    mesh = plsc.VectorSubcoreMesh(
        core_axis_name="c", subcore_axis_name="s", num_cores=NC, num_subcores=NS
    )

    @functools.partial(
        pl.kernel,
        out_type=jax.ShapeDtypeStruct((NC, N, D), jnp.float32),
        mesh=mesh,
        scratch_types=[
            [pltpu.VMEM((CH,), jnp.int32) for _ in range(4)],
            [pltpu.VMEM((CH,), jnp.int32) for _ in range(4)],
            [pltpu.VMEM((CH, D), jnp.float32) for _ in range(2)],
            pltpu.VMEM((ZR, D), jnp.float32),
            pltpu.VMEM_SHARED((N, D), jnp.float32),
            [pltpu.SemaphoreType.DMA for _ in range(4)],
            [pltpu.SemaphoreType.DMA for _ in range(4)],
            [pltpu.SemaphoreType.DMA for _ in range(2)],
        ],
    )
    def k(x_hbm, src_hbm, dst_hbm, out_hbm,
          sidx, didx, rows, zbuf, acc, sis, dis, sg):
        cid = lax.axis_index("c")
        sid = lax.axis_index("s")
        wid = sid * NC + cid
        start = wid * CPW
        nz = jnp.where(sid == NS - 1, (N // ZR) - 8 * (NS - 1), 8)

        zv = jnp.zeros((L,), jnp.float32)

        def zero_body(i, _):
            zbuf[i // (D // L), pl.ds((i % (D // L)) * L, L)] = zv
            return 0

        lax.fori_loop(0, ZR * (D // L), zero_body, 0)

        def init_body(j, _):
            r0 = pl.multiple_of((sid * 8 + j) * ZR, 8)
            pltpu.sync_copy(zbuf, acc.at[pl.ds(r0, ZR)])
            return 0

        lax.fori_loop(0, nz, init_body, 0)
        plsc.subcore_barrier()

        def issue_idx(c, q):
            pltpu.async_copy(src_hbm.at[c], sidx[q], sis[q])
            pltpu.async_copy(dst_hbm.at[c], didx[q], dis[q])

        def wait_idx(c, q):
            pltpu.make_async_copy(src_hbm.at[c], sidx[q], sis[q]).wait()
            pltpu.make_async_copy(dst_hbm.at[c], didx[q], dis[q]).wait()

        def issue_gather(q, b):
            pltpu.async_copy(x_hbm.at[sidx[q]], rows[b], sg[b])

        def wait_gather(q, b):
            pltpu.make_async_copy(x_hbm.at[sidx[q]], rows[b], sg[b]).wait()

        def scatter(q, b):
            pltpu.sync_copy(rows[b], acc.at[didx[q]], add=True)

        for q in range(4):
            issue_idx(start + q, q)
        wait_idx(start, 0)
        issue_gather(0, 0)

        def quad(j, _):
            for k4 in range(4):
                c = j + k4
                q, b = k4, k4 % 2
                qn, bn = (k4 + 1) % 4, (k4 + 1) % 2
                wait_idx(start + c + 1, qn)
                issue_gather(qn, bn)
                wait_gather(q, b)
                scatter(q, b)
                if k4 < 2:
                    issue_idx(start + c + 4, q)
                else:
                    @pl.when(c + 4 < CPW)
                    def _():
                        issue_idx(start + c + 4, q)
            return 0

        lax.fori_loop(0, (CPW - 2) // 4, lambda i, _: quad(i * 4, _), 0)

        wait_idx(start + CPW - 1, 1)
        issue_gather(1, 1)
        wait_gather(0, 0)
        scatter(0, 0)
        wait_gather(1, 1)
        scatter(1, 1)

        @pl.when(wid < NLEFT)
        def _():
            c = NW * CPW + wid
            issue_idx(c, 2)
            wait_idx(c, 2)
            issue_gather(2, 0)
            wait_gather(2, 0)
            scatter(2, 0)

        plsc.subcore_barrier()

        def drain_body(j, _):
            r0 = pl.multiple_of((sid * 8 + j) * ZR, 8)
            pltpu.sync_copy(acc.at[pl.ds(r0, ZR)], zbuf)
            pltpu.sync_copy(zbuf, out_hbm.at[cid, pl.ds(r0, ZR)])
            return 0

        lax.fori_loop(0, nz, drain_body, 0)

    return k(x, src2, dst2)


def _linear_kernel(act):
    def body(x_ref, p0_ref, p1_ref, w_ref, b_ref, o_ref):
        h = x_ref[...] + p0_ref[...] + p1_ref[...]
        z = jnp.dot(h, w_ref[...], preferred_element_type=jnp.float32)
        z = z + b_ref[...]
        if act == "relu":
            o_ref[...] = jnp.maximum(z, 0.0)
        else:
            m = jnp.max(z, axis=1, keepdims=True)
            e = jnp.exp(z - m)
            s = jnp.sum(e, axis=1, keepdims=True)
            o_ref[...] = (z - m) - jnp.log(s)

    return body


def _linear(x, parts, wt, b, act):
    BLK = 1000
    grid = (N // BLK,)
    row_spec = pl.BlockSpec((BLK, D), lambda i: (i, 0))
    full_spec = pl.BlockSpec((D, D), lambda i: (0, 0))
    bias_spec = pl.BlockSpec((1, D), lambda i: (0, 0))
    return pl.pallas_call(
        _linear_kernel(act),
        grid=grid,
        in_specs=[row_spec, row_spec, row_spec, full_spec, bias_spec],
        out_specs=row_spec,
        out_shape=jax.ShapeDtypeStruct((N, D), jnp.float32),
    )(x, parts[0], parts[1], wt, b)


def kernel(x, edge_index, W1, b1, W2, b2):
    src2 = edge_index[0].reshape(NCHUNKS, CH)
    dst2 = edge_index[1].reshape(NCHUNKS, CH)
    agg1 = _segment_sum_sc(x, src2, dst2)
    h1 = _linear(x, agg1, W1.T, b1.reshape(1, D), "relu")
    agg2 = _segment_sum_sc(h1, src2, dst2)
    return _linear(h1, agg2, W2.T, b2.reshape(1, D), "logsoftmax")


# speedup vs baseline: 12.2997x; 1.0023x over previous
import functools

import jax
import jax.numpy as jnp
from jax import lax
from jax.experimental import pallas as pl
from jax.experimental.pallas import tpu as pltpu
from jax.experimental.pallas import tpu_sc as plsc

N = 10000
E = 320000
D = 128

NC, NS, L = 2, 16, 16
NW = NC * NS
CH = 128
NCHUNKS = E // CH
CPW = NCHUNKS // NW
NLEFT = NCHUNKS - CPW * NW
ZR = 80


def _segment_sum_sc(x, src2, dst2):
    mesh = plsc.VectorSubcoreMesh(
        core_axis_name="c", subcore_axis_name="s", num_cores=NC, num_subcores=NS
    )

    @functools.partial(
        pl.kernel,
        out_type=jax.ShapeDtypeStruct((NC, N, D), jnp.float32),
        mesh=mesh,
        scratch_types=[
            [pltpu.VMEM((CH,), jnp.int32) for _ in range(4)],
            [pltpu.VMEM((CH,), jnp.int32) for _ in range(4)],
            [pltpu.VMEM((CH, D), jnp.float32) for _ in range(2)],
            pltpu.VMEM((ZR, D), jnp.float32),
            pltpu.VMEM_SHARED((N, D), jnp.float32),
            [pltpu.SemaphoreType.DMA for _ in range(4)],
            [pltpu.SemaphoreType.DMA for _ in range(4)],
            [pltpu.SemaphoreType.DMA for _ in range(2)],
            [pltpu.SemaphoreType.DMA for _ in range(2)],
        ],
    )
    def k(x_hbm, src_hbm, dst_hbm, out_hbm,
          sidx, didx, rows, zbuf, acc, sis, dis, sg, ss):
        cid = lax.axis_index("c")
        sid = lax.axis_index("s")
        wid = sid * NC + cid
        start = wid * CPW
        nz = jnp.where(sid == NS - 1, (N // ZR) - 8 * (NS - 1), 8)

        zv = jnp.zeros((L,), jnp.float32)

        def zero_body(i, _):
            zbuf[i // (D // L), pl.ds((i % (D // L)) * L, L)] = zv
            return 0

        lax.fori_loop(0, ZR * (D // L), zero_body, 0)

        def init_body(j, _):
            r0 = pl.multiple_of((sid * 8 + j) * ZR, 8)
            pltpu.sync_copy(zbuf, acc.at[pl.ds(r0, ZR)])
            return 0

        lax.fori_loop(0, nz, init_body, 0)
        plsc.subcore_barrier()

        def issue_idx(c, q):
            pltpu.async_copy(src_hbm.at[c], sidx[q], sis[q])
            pltpu.async_copy(dst_hbm.at[c], didx[q], dis[q])

        def wait_idx(c, q):
            pltpu.make_async_copy(src_hbm.at[c], sidx[q], sis[q]).wait()
            pltpu.make_async_copy(dst_hbm.at[c], didx[q], dis[q]).wait()

        def issue_gather(q, b):
            pltpu.async_copy(x_hbm.at[sidx[q]], rows[b], sg[b])

        def wait_gather(q, b):
            pltpu.make_async_copy(x_hbm.at[sidx[q]], rows[b], sg[b]).wait()

        def issue_scatter(q, b):
            pltpu.async_copy(rows[b], acc.at[didx[q]], ss[b], add=True)

        def wait_scatter(q, b):
            pltpu.make_async_copy(rows[b], acc.at[didx[q]], ss[b]).wait()

        for q in range(3):
            issue_idx(start + q, q)
        wait_idx(start, 0)
        issue_gather(0, 0)

        def chunk_body(c, k4, first, last):
            q, b = k4, k4 % 2
            qn, bn = (k4 + 1) % 4, (k4 + 1) % 2
            qp, bp = (k4 + 3) % 4, (k4 + 1) % 2
            if not first:
                wait_scatter(qp, bp)
                if last:
                    @pl.when(c + 3 < CPW)
                    def _():
                        issue_idx(start + c + 3, qp)
                else:
                    issue_idx(start + c + 3, qp)
            else:
                issue_idx(start + c + 3, (k4 + 3) % 4)
            wait_idx(start + c + 1, qn)
            issue_gather(qn, bn)
            wait_gather(q, b)
            issue_scatter(q, b)

        for k4 in range(4):
            chunk_body(k4, k4, first=(k4 == 0), last=False)

        def quad(i, _):
            j = 4 + i * 4
            for k4 in range(4):
                chunk_body(j + k4, k4, first=False, last=True)
            return 0

        lax.fori_loop(0, (CPW - 4 - 2) // 4, quad, 0)

        wait_scatter(3, 1)
        wait_idx(start + CPW - 1, 1)
        issue_gather(1, 1)
        wait_gather(0, 0)
        issue_scatter(0, 0)
        wait_scatter(0, 0)
        wait_gather(1, 1)
        issue_scatter(1, 1)
        wait_scatter(1, 1)

        @pl.when(wid < NLEFT)
        def _():
            c = NW * CPW + wid
            issue_idx(c, 2)
            wait_idx(c, 2)
            issue_gather(2, 0)
            wait_gather(2, 0)
            issue_scatter(2, 0)
            wait_scatter(2, 0)

        plsc.subcore_barrier()

        def drain_body(j, _):
            r0 = pl.multiple_of((sid * 8 + j) * ZR, 8)
            pltpu.sync_copy(acc.at[pl.ds(r0, ZR)], out_hbm.at[cid, pl.ds(r0, ZR)])
            return 0

        lax.fori_loop(0, nz, drain_body, 0)

    return k(x, src2, dst2)


def _linear_kernel(act):
    def body(x_ref, p0_ref, p1_ref, w_ref, b_ref, o_ref):
        h = x_ref[...] + p0_ref[...] + p1_ref[...]
        z = jnp.dot(h, w_ref[...], preferred_element_type=jnp.float32)
        z = z + b_ref[...]
        if act == "relu":
            o_ref[...] = jnp.maximum(z, 0.0)
        else:
            m = jnp.max(z, axis=1, keepdims=True)
            e = jnp.exp(z - m)
            s = jnp.sum(e, axis=1, keepdims=True)
            o_ref[...] = (z - m) - jnp.log(s)

    return body


def _linear(x, parts, wt, b, act):
    BLK = 1000
    grid = (N // BLK,)
    row_spec = pl.BlockSpec((BLK, D), lambda i: (i, 0))
    full_spec = pl.BlockSpec((D, D), lambda i: (0, 0))
    bias_spec = pl.BlockSpec((1, D), lambda i: (0, 0))
    return pl.pallas_call(
        _linear_kernel(act),
        grid=grid,
        in_specs=[row_spec, row_spec, row_spec, full_spec, bias_spec],
        out_specs=row_spec,
        out_shape=jax.ShapeDtypeStruct((N, D), jnp.float32),
    )(x, parts[0], parts[1], wt, b)


def kernel(x, edge_index, W1, b1, W2, b2):
    src2 = edge_index[0].reshape(NCHUNKS, CH)
    dst2 = edge_index[1].reshape(NCHUNKS, CH)
    agg1 = _segment_sum_sc(x, src2, dst2)
    h1 = _linear(x, agg1, W1.T, b1.reshape(1, D), "relu")
    agg2 = _segment_sum_sc(h1, src2, dst2)
    return _linear(h1, agg2, W2.T, b2.reshape(1, D), "logsoftmax")


# speedup vs baseline: 13.4335x; 1.0922x over previous
import functools

import jax
import jax.numpy as jnp
from jax import lax
from jax.experimental import pallas as pl
from jax.experimental.pallas import tpu as pltpu
from jax.experimental.pallas import tpu_sc as plsc

N = 10000
E = 320000
D = 128

NC, NS, L = 2, 16, 16
NW = NC * NS
CH = 128
NCHUNKS = E // CH
CPW = NCHUNKS // NW
NLEFT = NCHUNKS - CPW * NW
ZR = 80


def _segment_sum_sc(x, e2):
    mesh = plsc.VectorSubcoreMesh(
        core_axis_name="c", subcore_axis_name="s", num_cores=NC, num_subcores=NS
    )

    @functools.partial(
        pl.kernel,
        out_type=jax.ShapeDtypeStruct((NC, N, D), jnp.float32),
        mesh=mesh,
        scratch_types=[
            [pltpu.VMEM((CH,), jnp.int32) for _ in range(4)],
            [pltpu.VMEM((CH,), jnp.int32) for _ in range(4)],
            [pltpu.VMEM((CH, D), jnp.float32) for _ in range(2)],
            pltpu.VMEM((ZR, D), jnp.float32),
            pltpu.VMEM_SHARED((N, D), jnp.float32),
            [pltpu.SemaphoreType.DMA for _ in range(4)],
            [pltpu.SemaphoreType.DMA for _ in range(4)],
            [pltpu.SemaphoreType.DMA for _ in range(2)],
            [pltpu.SemaphoreType.DMA for _ in range(2)],
        ],
    )
    def k(x_hbm, e_hbm, out_hbm,
          sidx, didx, rows, zbuf, acc, sis, dis, sg, ss):
        src_hbm = e_hbm.at[0]
        dst_hbm = e_hbm.at[1]
        cid = lax.axis_index("c")
        sid = lax.axis_index("s")
        wid = sid * NC + cid
        start = wid * CPW
        nz = jnp.where(sid == NS - 1, (N // ZR) - 8 * (NS - 1), 8)

        zv = jnp.zeros((L,), jnp.float32)

        def zero_body(i, _):
            zbuf[i // (D // L), pl.ds((i % (D // L)) * L, L)] = zv
            return 0

        lax.fori_loop(0, ZR * (D // L), zero_body, 0)

        def init_body(j, _):
            r0 = pl.multiple_of((sid * 8 + j) * ZR, 8)
            pltpu.sync_copy(zbuf, acc.at[pl.ds(r0, ZR)])
            return 0

        lax.fori_loop(0, nz, init_body, 0)
        plsc.subcore_barrier()

        def issue_idx(c, q):
            pltpu.async_copy(src_hbm.at[c], sidx[q], sis[q])
            pltpu.async_copy(dst_hbm.at[c], didx[q], dis[q])

        def wait_idx(c, q):
            pltpu.make_async_copy(src_hbm.at[c], sidx[q], sis[q]).wait()
            pltpu.make_async_copy(dst_hbm.at[c], didx[q], dis[q]).wait()

        def issue_gather(q, b):
            pltpu.async_copy(x_hbm.at[sidx[q]], rows[b], sg[b])

        def wait_gather(q, b):
            pltpu.make_async_copy(x_hbm.at[sidx[q]], rows[b], sg[b]).wait()

        def issue_scatter(q, b):
            pltpu.async_copy(rows[b], acc.at[didx[q]], ss[b], add=True)

        def wait_scatter(q, b):
            pltpu.make_async_copy(rows[b], acc.at[didx[q]], ss[b]).wait()

        for q in range(3):
            issue_idx(start + q, q)
        wait_idx(start, 0)
        issue_gather(0, 0)

        def chunk_body(c, k4, first, last):
            q, b = k4, k4 % 2
            qn, bn = (k4 + 1) % 4, (k4 + 1) % 2
            qp, bp = (k4 + 3) % 4, (k4 + 1) % 2
            if not first:
                wait_scatter(qp, bp)
                if last:
                    @pl.when(c + 3 < CPW)
                    def _():
                        issue_idx(start + c + 3, qp)
                else:
                    issue_idx(start + c + 3, qp)
            else:
                issue_idx(start + c + 3, (k4 + 3) % 4)
            wait_idx(start + c + 1, qn)
            issue_gather(qn, bn)
            wait_gather(q, b)
            issue_scatter(q, b)

        for k4 in range(4):
            chunk_body(k4, k4, first=(k4 == 0), last=False)

        def quad(i, _):
            j = 4 + i * 4
            for k4 in range(4):
                chunk_body(j + k4, k4, first=False, last=True)
            return 0

        lax.fori_loop(0, (CPW - 4 - 2) // 4, quad, 0)

        wait_scatter(3, 1)
        wait_idx(start + CPW - 1, 1)
        issue_gather(1, 1)
        wait_gather(0, 0)
        issue_scatter(0, 0)
        wait_scatter(0, 0)
        wait_gather(1, 1)
        issue_scatter(1, 1)
        wait_scatter(1, 1)

        @pl.when(wid < NLEFT)
        def _():
            c = NW * CPW + wid
            issue_idx(c, 2)
            wait_idx(c, 2)
            issue_gather(2, 0)
            wait_gather(2, 0)
            issue_scatter(2, 0)
            wait_scatter(2, 0)

        plsc.subcore_barrier()

        def drain_body(j, _):
            r0 = pl.multiple_of((sid * 8 + j) * ZR, 8)
            pltpu.sync_copy(acc.at[pl.ds(r0, ZR)], out_hbm.at[cid, pl.ds(r0, ZR)])
            return 0

        lax.fori_loop(0, nz, drain_body, 0)

    return k(x, e2)


def _linear_kernel(act):
    def body(x_ref, p_ref, w_ref, b_ref, o_ref):
        h = x_ref[...] + p_ref[0] + p_ref[1]
        z = jnp.dot(h, w_ref[...], preferred_element_type=jnp.float32)
        z = z + b_ref[...]
        if act == "relu":
            o_ref[...] = jnp.maximum(z, 0.0)
        else:
            m = jnp.max(z, axis=1, keepdims=True)
            e = jnp.exp(z - m)
            s = jnp.sum(e, axis=1, keepdims=True)
            o_ref[...] = (z - m) - jnp.log(s)

    return body


def _linear(x, parts, wt, b, act):
    BLK = 1000
    grid = (N // BLK,)
    row_spec = pl.BlockSpec((BLK, D), lambda i: (i, 0))
    parts_spec = pl.BlockSpec((2, BLK, D), lambda i: (0, i, 0))
    full_spec = pl.BlockSpec((D, D), lambda i: (0, 0))
    bias_spec = pl.BlockSpec((1, D), lambda i: (0, 0))
    return pl.pallas_call(
        _linear_kernel(act),
        grid=grid,
        in_specs=[row_spec, parts_spec, full_spec, bias_spec],
        out_specs=row_spec,
        out_shape=jax.ShapeDtypeStruct((N, D), jnp.float32),
    )(x, parts, wt, b)


def kernel(x, edge_index, W1, b1, W2, b2):
    e2 = edge_index.reshape(2, NCHUNKS, CH)
    agg1 = _segment_sum_sc(x, e2)
    h1 = _linear(x, agg1, W1.T, b1.reshape(1, D), "relu")
    agg2 = _segment_sum_sc(h1, e2)
    return _linear(h1, agg2, W2.T, b2.reshape(1, D), "logsoftmax")


# speedup vs baseline: 13.5984x; 1.0123x over previous
import functools

import jax
import jax.numpy as jnp
from jax import lax
from jax.experimental import pallas as pl
from jax.experimental.pallas import tpu as pltpu
from jax.experimental.pallas import tpu_sc as plsc

N = 10000
E = 320000
D = 128

NC, NS, L = 2, 16, 16
NW = NC * NS
CH = 128
NCHUNKS = E // CH
CPW = NCHUNKS // NW
NLEFT = NCHUNKS - CPW * NW
ZR = 80


def _segment_sum_sc(x, e2):
    mesh = plsc.VectorSubcoreMesh(
        core_axis_name="c", subcore_axis_name="s", num_cores=NC, num_subcores=NS
    )

    @functools.partial(
        pl.kernel,
        out_type=jax.ShapeDtypeStruct((NC, N, D), jnp.float32),
        mesh=mesh,
        scratch_types=[
            [pltpu.VMEM((CH,), jnp.int32) for _ in range(4)],
            [pltpu.VMEM((CH,), jnp.int32) for _ in range(4)],
            [pltpu.VMEM((CH, D), jnp.float32) for _ in range(2)],
            pltpu.VMEM((ZR, D), jnp.float32),
            pltpu.VMEM_SHARED((N, D), jnp.float32),
            [pltpu.SemaphoreType.DMA for _ in range(4)],
            [pltpu.SemaphoreType.DMA for _ in range(4)],
            [pltpu.SemaphoreType.DMA for _ in range(2)],
            [pltpu.SemaphoreType.DMA for _ in range(2)],
        ],
    )
    def k(x_hbm, e_hbm, out_hbm,
          sidx, didx, rows, zbuf, acc, sis, dis, sg, ss):
        src_hbm = e_hbm.at[0]
        dst_hbm = e_hbm.at[1]
        cid = lax.axis_index("c")
        sid = lax.axis_index("s")
        wid = sid * NC + cid
        start = wid * CPW
        nz = jnp.where(sid == NS - 1, (N // ZR) - 8 * (NS - 1), 8)

        zv = jnp.zeros((L,), jnp.float32)

        def zero_body(i, _):
            zbuf[i // (D // L), pl.ds((i % (D // L)) * L, L)] = zv
            return 0

        lax.fori_loop(0, ZR * (D // L), zero_body, 0)

        def init_body(j, _):
            r0 = pl.multiple_of((sid * 8 + j) * ZR, 8)
            pltpu.sync_copy(zbuf, acc.at[pl.ds(r0, ZR)])
            return 0

        lax.fori_loop(0, nz, init_body, 0)
        plsc.subcore_barrier()

        def issue_idx(c, q):
            pltpu.async_copy(src_hbm.at[pl.ds(c * CH, CH)], sidx[q], sis[q])
            pltpu.async_copy(dst_hbm.at[pl.ds(c * CH, CH)], didx[q], dis[q])

        def wait_idx(c, q):
            pltpu.make_async_copy(src_hbm.at[pl.ds(c * CH, CH)], sidx[q], sis[q]).wait()
            pltpu.make_async_copy(dst_hbm.at[pl.ds(c * CH, CH)], didx[q], dis[q]).wait()

        def issue_gather(q, b):
            pltpu.async_copy(x_hbm.at[sidx[q]], rows[b], sg[b])

        def wait_gather(q, b):
            pltpu.make_async_copy(x_hbm.at[sidx[q]], rows[b], sg[b]).wait()

        def issue_scatter(q, b):
            pltpu.async_copy(rows[b], acc.at[didx[q]], ss[b], add=True)

        def wait_scatter(q, b):
            pltpu.make_async_copy(rows[b], acc.at[didx[q]], ss[b]).wait()

        for q in range(3):
            issue_idx(start + q, q)
        wait_idx(start, 0)
        issue_gather(0, 0)

        def chunk_body(c, k4, first, last):
            q, b = k4, k4 % 2
            qn, bn = (k4 + 1) % 4, (k4 + 1) % 2
            qp, bp = (k4 + 3) % 4, (k4 + 1) % 2
            if not first:
                wait_scatter(qp, bp)
                if last:
                    @pl.when(c + 3 < CPW)
                    def _():
                        issue_idx(start + c + 3, qp)
                else:
                    issue_idx(start + c + 3, qp)
            else:
                issue_idx(start + c + 3, (k4 + 3) % 4)
            wait_idx(start + c + 1, qn)
            issue_gather(qn, bn)
            wait_gather(q, b)
            issue_scatter(q, b)

        for k4 in range(4):
            chunk_body(k4, k4, first=(k4 == 0), last=False)

        def quad(i, _):
            j = 4 + i * 4
            for k4 in range(4):
                chunk_body(j + k4, k4, first=False, last=True)
            return 0

        lax.fori_loop(0, (CPW - 4 - 2) // 4, quad, 0)

        wait_scatter(3, 1)
        wait_idx(start + CPW - 1, 1)
        issue_gather(1, 1)
        wait_gather(0, 0)
        issue_scatter(0, 0)
        wait_scatter(0, 0)
        wait_gather(1, 1)
        issue_scatter(1, 1)
        wait_scatter(1, 1)

        @pl.when(wid < NLEFT)
        def _():
            c = NW * CPW + wid
            issue_idx(c, 2)
            wait_idx(c, 2)
            issue_gather(2, 0)
            wait_gather(2, 0)
            issue_scatter(2, 0)
            wait_scatter(2, 0)

        plsc.subcore_barrier()

        def drain_body(j, _):
            r0 = pl.multiple_of((sid * 8 + j) * ZR, 8)
            pltpu.sync_copy(acc.at[pl.ds(r0, ZR)], out_hbm.at[cid, pl.ds(r0, ZR)])
            return 0

        lax.fori_loop(0, nz, drain_body, 0)

    return k(x, e2)


def _linear_kernel(act):
    def body(x_ref, p_ref, w_ref, b_ref, o_ref):
        h = x_ref[...] + p_ref[0] + p_ref[1]
        z = jnp.dot(h, w_ref[...], preferred_element_type=jnp.float32)
        z = z + b_ref[...]
        if act == "relu":
            o_ref[...] = jnp.maximum(z, 0.0)
        else:
            m = jnp.max(z, axis=1, keepdims=True)
            e = jnp.exp(z - m)
            s = jnp.sum(e, axis=1, keepdims=True)
            o_ref[...] = (z - m) - jnp.log(s)

    return body


def _linear(x, parts, wt, b, act):
    BLK = 1000
    grid = (N // BLK,)
    row_spec = pl.BlockSpec((BLK, D), lambda i: (i, 0))
    parts_spec = pl.BlockSpec((2, BLK, D), lambda i: (0, i, 0))
    full_spec = pl.BlockSpec((D, D), lambda i: (0, 0))
    bias_spec = pl.BlockSpec((1, D), lambda i: (0, 0))
    return pl.pallas_call(
        _linear_kernel(act),
        grid=grid,
        in_specs=[row_spec, parts_spec, full_spec, bias_spec],
        out_specs=row_spec,
        out_shape=jax.ShapeDtypeStruct((N, D), jnp.float32),
    )(x, parts, wt, b)


def kernel(x, edge_index, W1, b1, W2, b2):
    agg1 = _segment_sum_sc(x, edge_index)
    h1 = _linear(x, agg1, W1.T, b1.reshape(1, D), "relu")
    agg2 = _segment_sum_sc(h1, edge_index)
    return _linear(h1, agg2, W2.T, b2.reshape(1, D), "logsoftmax")


# speedup vs baseline: 13.7462x; 1.0109x over previous
import functools

import jax
import jax.numpy as jnp
from jax import lax
from jax.experimental import pallas as pl
from jax.experimental.pallas import tpu as pltpu
from jax.experimental.pallas import tpu_sc as plsc

N = 10000
E = 320000
D = 128

NC, NS, L = 2, 16, 16
NW = NC * NS
CH = 128
NCHUNKS = E // CH
CPW = NCHUNKS // NW
NLEFT = NCHUNKS - CPW * NW
ZR = 80


def _segment_sum_sc(x, e2):
    mesh = plsc.VectorSubcoreMesh(
        core_axis_name="c", subcore_axis_name="s", num_cores=NC, num_subcores=NS
    )

    @functools.partial(
        pl.kernel,
        out_type=jax.ShapeDtypeStruct((NC, N, D), jnp.float32),
        mesh=mesh,
        scratch_types=[
            [pltpu.VMEM((CH,), jnp.int32) for _ in range(4)],
            [pltpu.VMEM((CH,), jnp.int32) for _ in range(4)],
            [pltpu.VMEM((CH, D), jnp.float32) for _ in range(2)],
            pltpu.VMEM((ZR, D), jnp.float32),
            pltpu.VMEM_SHARED((N, D), jnp.float32),
            [pltpu.SemaphoreType.DMA for _ in range(4)],
            [pltpu.SemaphoreType.DMA for _ in range(4)],
            [pltpu.SemaphoreType.DMA for _ in range(2)],
            [pltpu.SemaphoreType.DMA for _ in range(2)],
            [pltpu.SemaphoreType.DMA for _ in range(8)],
        ],
    )
    def k(x_hbm, e_hbm, out_hbm,
          sidx, didx, rows, zbuf, acc, sis, dis, sg, ss, ds):
        src_hbm = e_hbm.at[0]
        dst_hbm = e_hbm.at[1]
        cid = lax.axis_index("c")
        sid = lax.axis_index("s")
        wid = sid * NC + cid
        start = wid * CPW
        nz = jnp.where(sid == NS - 1, (N // ZR) - 8 * (NS - 1), 8)

        def issue_idx(c, q):
            pltpu.async_copy(src_hbm.at[pl.ds(c * CH, CH)], sidx[q], sis[q])
            pltpu.async_copy(dst_hbm.at[pl.ds(c * CH, CH)], didx[q], dis[q])

        def wait_idx(c, q):
            pltpu.make_async_copy(src_hbm.at[pl.ds(c * CH, CH)], sidx[q], sis[q]).wait()
            pltpu.make_async_copy(dst_hbm.at[pl.ds(c * CH, CH)], didx[q], dis[q]).wait()

        def issue_gather(q, b):
            pltpu.async_copy(x_hbm.at[sidx[q]], rows[b], sg[b])

        def wait_gather(q, b):
            pltpu.make_async_copy(x_hbm.at[sidx[q]], rows[b], sg[b]).wait()

        def issue_scatter(q, b):
            pltpu.async_copy(rows[b], acc.at[didx[q]], ss[b], add=True)

        def wait_scatter(q, b):
            pltpu.make_async_copy(rows[b], acc.at[didx[q]], ss[b]).wait()

        for q in range(3):
            issue_idx(start + q, q)

        zv = jnp.zeros((L,), jnp.float32)

        def zero_body(i, _):
            zbuf[i // (D // L), pl.ds((i % (D // L)) * L, L)] = zv
            return 0

        lax.fori_loop(0, ZR * (D // L), zero_body, 0)

        for j in range(8):
            @pl.when(j < nz)
            def _():
                r0 = pl.multiple_of((sid * 8 + j) * ZR, 8)
                pltpu.async_copy(zbuf, acc.at[pl.ds(r0, ZR)], ds[j])

        wait_idx(start, 0)
        issue_gather(0, 0)

        for j in range(8):
            @pl.when(j < nz)
            def _():
                r0 = pl.multiple_of((sid * 8 + j) * ZR, 8)
                pltpu.make_async_copy(zbuf, acc.at[pl.ds(r0, ZR)], ds[j]).wait()

        plsc.subcore_barrier()


        def chunk_body(c, k4, first, last):
            q, b = k4, k4 % 2
            qn, bn = (k4 + 1) % 4, (k4 + 1) % 2
            qp, bp = (k4 + 3) % 4, (k4 + 1) % 2
            if not first:
                wait_scatter(qp, bp)
                if last:
                    @pl.when(c + 3 < CPW)
                    def _():
                        issue_idx(start + c + 3, qp)
                else:
                    issue_idx(start + c + 3, qp)
            else:
                issue_idx(start + c + 3, (k4 + 3) % 4)
            wait_idx(start + c + 1, qn)
            issue_gather(qn, bn)
            wait_gather(q, b)
            issue_scatter(q, b)

        for k4 in range(4):
            chunk_body(k4, k4, first=(k4 == 0), last=False)

        def quad(i, _):
            j = 4 + i * 4
            for k4 in range(4):
                chunk_body(j + k4, k4, first=False, last=True)
            return 0

        lax.fori_loop(0, (CPW - 4 - 2) // 4, quad, 0)

        wait_scatter(3, 1)
        wait_idx(start + CPW - 1, 1)
        issue_gather(1, 1)
        wait_gather(0, 0)
        issue_scatter(0, 0)
        wait_scatter(0, 0)
        wait_gather(1, 1)
        issue_scatter(1, 1)
        wait_scatter(1, 1)

        @pl.when(wid < NLEFT)
        def _():
            c = NW * CPW + wid
            issue_idx(c, 2)
            wait_idx(c, 2)
            issue_gather(2, 0)
            wait_gather(2, 0)
            issue_scatter(2, 0)
            wait_scatter(2, 0)

        plsc.subcore_barrier()

        for j in range(8):
            @pl.when(j < nz)
            def _():
                r0 = pl.multiple_of((sid * 8 + j) * ZR, 8)
                pltpu.async_copy(
                    acc.at[pl.ds(r0, ZR)], out_hbm.at[cid, pl.ds(r0, ZR)], ds[j]
                )

        for j in range(8):
            @pl.when(j < nz)
            def _():
                r0 = pl.multiple_of((sid * 8 + j) * ZR, 8)
                pltpu.make_async_copy(
                    acc.at[pl.ds(r0, ZR)], out_hbm.at[cid, pl.ds(r0, ZR)], ds[j]
                ).wait()

    return k(x, e2)


def _linear_kernel(act):
    def body(x_ref, p_ref, w_ref, b_ref, o_ref):
        h = x_ref[...] + p_ref[0] + p_ref[1]
        z = jnp.dot(h, w_ref[...], preferred_element_type=jnp.float32)
        z = z + b_ref[...]
        if act == "relu":
            o_ref[...] = jnp.maximum(z, 0.0)
        else:
            m = jnp.max(z, axis=1, keepdims=True)
            e = jnp.exp(z - m)
            s = jnp.sum(e, axis=1, keepdims=True)
            o_ref[...] = (z - m) - jnp.log(s)

    return body


def _linear(x, parts, wt, b, act):
    BLK = 1000
    grid = (N // BLK,)
    row_spec = pl.BlockSpec((BLK, D), lambda i: (i, 0))
    parts_spec = pl.BlockSpec((2, BLK, D), lambda i: (0, i, 0))
    full_spec = pl.BlockSpec((D, D), lambda i: (0, 0))
    bias_spec = pl.BlockSpec((1, D), lambda i: (0, 0))
    return pl.pallas_call(
        _linear_kernel(act),
        grid=grid,
        in_specs=[row_spec, parts_spec, full_spec, bias_spec],
        out_specs=row_spec,
        out_shape=jax.ShapeDtypeStruct((N, D), jnp.float32),
    )(x, parts, wt, b)


def kernel(x, edge_index, W1, b1, W2, b2):
    agg1 = _segment_sum_sc(x, edge_index)
    h1 = _linear(x, agg1, W1.T, b1.reshape(1, D), "relu")
    agg2 = _segment_sum_sc(h1, edge_index)
    return _linear(h1, agg2, W2.T, b2.reshape(1, D), "logsoftmax")
